# baseline (speedup 1.0000x reference)
import functools

import jax
import jax.numpy as jnp
from jax import lax
from jax.experimental import pallas as pl
from jax.experimental.pallas import tpu as pltpu
from jax.experimental.pallas import tpu_sc as plsc

N = 10000
E = 320000
D = 128
DE = 16
DH = 256
G = 64

NC = 2
NS = 16
NW = NC * NS
GCH = 40
K = 5
GRP = K * GCH
NPW = 624
NTAIL = N - NS * NPW

EA = 192000
EB = E - EA

TE = 2000

K2 = 2
GRP2 = K2 * GCH
IDXC = 100
NGB = IDXC // K2

_sc_mesh = plsc.VectorSubcoreMesh(core_axis_name="c", subcore_axis_name="s",
                                  num_cores=NC, num_subcores=NS)


def _make_gather(ne):
    epw = ne // NW
    ndma = epw // GCH
    ng = epw // GRP
    assert ng * GRP == epw and ng % 2 == 0

    @functools.partial(
        pl.kernel,
        out_type=(jax.ShapeDtypeStruct((ne, D), jnp.float32),
                  jax.ShapeDtypeStruct((ne, D), jnp.float32)),
        mesh=_sc_mesh,
        scratch_types=[
            pltpu.VMEM((ndma, GCH), jnp.int32),
            pltpu.VMEM((GRP, D), jnp.float32),
            pltpu.VMEM((GRP, D), jnp.float32),
            pltpu.SemaphoreType.DMA,
            pltpu.SemaphoreType.DMA,
            pltpu.SemaphoreType.DMA,
            pltpu.SemaphoreType.DMA,
        ],
    )
    def gather(h_hbm, src_hbm, dst_hbm, hs_hbm, hd_hbm,
               idx_v, rows0, rows1, sem0, sem1, wsem0, wsem1):
        cid = lax.axis_index("c")
        sid = lax.axis_index("s")
        wid = sid * NC + cid
        base = wid * epw

        for ind_hbm, out_hbm in ((src_hbm, hs_hbm), (dst_hbm, hd_hbm)):
            pltpu.sync_copy(ind_hbm.at[wid], idx_v)

            def fire(g, rows, sem):
                for b in range(K):
                    pltpu.async_copy(h_hbm.at[idx_v.at[g * K + b]],
                                     rows.at[pl.ds(b * GCH, GCH)], sem)

            def drain(rows, sem):
                for b in range(K):
                    pltpu.make_async_copy(h_hbm.at[pl.ds(0, GCH)],
                                          rows.at[pl.ds(b * GCH, GCH)],
                                          sem).wait()

            def process(g, rows, sem, wsem, orows, osem, owsem, fire_guard):
                @pl.when(g >= 1)
                def _():
                    pltpu.make_async_copy(orows, out_hbm.at[pl.ds(base, GRP)],
                                          owsem).wait()
                if fire_guard is None:
                    fire(g + 1, orows, osem)
                else:
                    @pl.when(fire_guard)
                    def _():
                        fire(g + 1, orows, osem)
                drain(rows, sem)
                pltpu.async_copy(rows, out_hbm.at[pl.ds(base + g * GRP, GRP)],
                                 wsem)

            fire(0, rows0, sem0)

            @pl.loop(0, ng, step=2)
            def _(g):
                process(g, rows0, sem0, wsem0, rows1, sem1, wsem1, None)
                process(g + 1, rows1, sem1, wsem1, rows0, sem0, wsem0,
                        g + 2 < ng)

            pltpu.make_async_copy(rows1, out_hbm.at[pl.ds(base, GRP)],
                                  wsem1).wait()

    return gather


_gather_a = _make_gather(EA)
_gather_b = _make_gather(EB)


NHALF = N // 2
NPN = NHALF + 8
NPW2 = 312


def _make_scatter(ne, nblk):
    epw2 = ne // NS
    assert nblk * IDXC * GCH == epw2

    @functools.partial(
        pl.kernel,
        out_type=jax.ShapeDtypeStruct((N, D), jnp.float32),
        mesh=_sc_mesh,
        scratch_types=[
            pltpu.VMEM((IDXC, GCH), jnp.int32),
            pltpu.VMEM((GRP2, D), jnp.float32),
            pltpu.VMEM((GRP2, D), jnp.float32),
            pltpu.VMEM_SHARED((NPN, D), jnp.float32),
            pltpu.SemaphoreType.DMA,
            pltpu.SemaphoreType.DMA,
            pltpu.SemaphoreType.DMA,
            pltpu.SemaphoreType.DMA,
        ],
    )
    def scatter(m_hbm, dsti_hbm, zeros_hbm, out_hbm,
                idx_v, rows0, rows1, acc, sem0, sem1, ssem0, ssem1):
        cid = lax.axis_index("c")
        sid = lax.axis_index("s")

        pltpu.sync_copy(zeros_hbm.at[pl.ds(sid * NPW2, NPW2)],
                        acc.at[pl.ds(sid * NPW2, NPW2)])

        @pl.when(sid == NS - 1)
        def _():
            pltpu.sync_copy(zeros_hbm.at[pl.ds(NS * NPW2, NPN - NS * NPW2)],
                            acc.at[pl.ds(NS * NPW2, NPN - NS * NPW2)])

        plsc.subcore_barrier()

        def fire(bbase, g, rows, sem):
            pltpu.async_copy(m_hbm.at[pl.ds(bbase + g * GRP2, GRP2)],
                             rows, sem)

        def drain_adds(orows, ssem):
            for b in range(K2):
                pltpu.make_async_copy(orows.at[pl.ds(b * GCH, GCH)],
                                      acc.at[pl.ds(0, GCH)], ssem).wait()

        def process(bbase, g, rows, sem, ssem_cur, orows, osem, ssem_opp,
                    fire_guard):
            @pl.when(g >= 1)
            def _():
                drain_adds(orows, ssem_opp)
            if fire_guard is None:
                fire(bbase, g + 1, orows, osem)
            else:
                @pl.when(fire_guard)
                def _():
                    fire(bbase, g + 1, orows, osem)
            pltpu.make_async_copy(m_hbm.at[pl.ds(0, GRP2)], rows,
                                  sem).wait()
            for b in range(K2):
                pltpu.async_copy(rows.at[pl.ds(b * GCH, GCH)],
                                 acc.at[idx_v.at[g * K2 + b]], ssem_cur,
                                 add=True)

        for ib in range(nblk):
            pltpu.sync_copy(dsti_hbm.at[cid].at[sid].at[ib], idx_v)
            bbase = sid * epw2 + ib * (IDXC * GCH)
            fire(bbase, 0, rows0, sem0)

            @pl.loop(0, NGB, step=2)
            def _(g):
                process(bbase, g, rows0, sem0, ssem0, rows1, sem1, ssem1,
                        None)
                process(bbase, g + 1, rows1, sem1, ssem1, rows0, sem0,
                        ssem0, g + 2 < NGB)

            drain_adds(rows1, ssem1)

        plsc.subcore_barrier()

        pltpu.sync_copy(acc.at[pl.ds(sid * NPW2, NPW2)],
                        out_hbm.at[pl.ds(cid * NHALF + sid * NPW2, NPW2)])

        @pl.when(sid == NS - 1)
        def _():
            pltpu.sync_copy(acc.at[pl.ds(NS * NPW2, NHALF - NS * NPW2)],
                            out_hbm.at[pl.ds(cid * NHALF + NS * NPW2,
                                             NHALF - NS * NPW2)])

    return scatter


_scatter_a = _make_scatter(EA, 3)
_scatter_b = _make_scatter(EB, 2)


def _edge_mlp_body(hd_ref, hs_ref, ea_ref, w1d_ref, w1s_ref, w1e_ref, b1_ref,
                   w2_ref, b2_ref, m_ref):
    t = (jnp.dot(hd_ref[...].astype(jnp.bfloat16), w1d_ref[...],
                 preferred_element_type=jnp.float32)
         + jnp.dot(hs_ref[...].astype(jnp.bfloat16), w1s_ref[...],
                   preferred_element_type=jnp.float32)
         + jnp.dot(ea_ref[...], w1e_ref[...], preferred_element_type=jnp.float32)
         + b1_ref[...])
    t = t * lax.logistic(t)
    m_ref[...] = jnp.dot(t.astype(jnp.bfloat16), w2_ref[...],
                         preferred_element_type=jnp.float32) + b2_ref[...]


def _edge_mlp(hd, hs, ea, W1d, W1s, W1e, b1, W2, b2):
    ne = hd.shape[0]
    grid = (ne // TE,)
    full = lambda shape: pl.BlockSpec(shape, lambda i: (0, 0))
    return pl.pallas_call(
        _edge_mlp_body,
        grid=grid,
        in_specs=[
            pl.BlockSpec((TE, D), lambda i: (i, 0)),
            pl.BlockSpec((TE, D), lambda i: (i, 0)),
            pl.BlockSpec((TE, DE), lambda i: (i, 0)),
            full((D, DH)), full((D, DH)), full((DE, DH)), full((1, DH)),
            full((DH, D)), full((1, D)),
        ],
        out_specs=pl.BlockSpec((TE, D), lambda i: (i, 0)),
        out_shape=jax.ShapeDtypeStruct((ne, D), jnp.float32),
    )(hd, hs, ea, W1d, W1s, W1e, b1, W2, b2)


def _update_norm_body(h_ref, a0_ref, a1_ref, wu1h_ref, wu1a_ref, bu1_ref,
                      wu2_ref, bu2_ref, batch_ref, gnw_ref, gnb_ref, gna_ref,
                      out_ref):
    h = h_ref[...]
    agg = a0_ref[...] + a1_ref[...]
    t = (jnp.dot(h, wu1h_ref[...], preferred_element_type=jnp.float32)
         + jnp.dot(agg, wu1a_ref[...], preferred_element_type=jnp.float32)
         + bu1_ref[...])
    t = t * lax.logistic(t)
    dh = jnp.dot(t, wu2_ref[...], preferred_element_type=jnp.float32) + bu2_ref[...]
    h2 = h + dh
    oh = (batch_ref[...] == lax.broadcasted_iota(jnp.int32, (1, G), 1)).astype(jnp.float32)
    counts = jnp.maximum(jnp.sum(oh, axis=0, keepdims=True), 1.0)
    inv_counts = jnp.reshape(1.0 / counts, (G, 1))
    seg = lax.dot_general(oh, h2, (((0,), (0,)), ((), ())),
                          preferred_element_type=jnp.float32)
    mean = seg * inv_counts
    out = h2 - gna_ref[...] * jnp.dot(oh, mean, preferred_element_type=jnp.float32)
    var = lax.dot_general(oh, out * out, (((0,), (0,)), ((), ())),
                          preferred_element_type=jnp.float32) * inv_counts
    inv = lax.rsqrt(var + 1e-5)
    out = out * jnp.dot(oh, inv, preferred_element_type=jnp.float32)
    out_ref[...] = gnw_ref[...] * out + gnb_ref[...]


def _update_norm(h, agg0, agg1, Wu1h, Wu1a, bu1, Wu2, bu2, batch2d, gnw, gnb, gna):
    return pl.pallas_call(
        _update_norm_body,
        out_shape=jax.ShapeDtypeStruct((N, D), jnp.float32),
    )(h, agg0, agg1, Wu1h, Wu1a, bu1, Wu2, bu2, batch2d, gnw, gnb, gna)


def kernel(h, edge_index, edge_attr, batch,
           W_msg1, b_msg1, W_msg2, b_msg2,
           W_upd1, b_upd1, W_upd2, b_upd2,
           gn_weight, gn_bias, gn_alpha):
    src = edge_index[0].astype(jnp.int32)
    dst = edge_index[1].astype(jnp.int32)
    W1d = W_msg1[:D].astype(jnp.bfloat16)
    W1s = W_msg1[D:2 * D].astype(jnp.bfloat16)
    W1e = W_msg1[2 * D:].astype(jnp.bfloat16)
    W2b = W_msg2.astype(jnp.bfloat16)
    eab = edge_attr.astype(jnp.bfloat16)
    b1 = b_msg1.reshape(1, DH)
    b2 = b_msg2.reshape(1, D)
    Wu1h = W_upd1[:D]
    Wu1a = W_upd1[D:]
    bu1 = b_upd1.reshape(1, DH)
    bu2 = b_upd2.reshape(1, D)
    gnw = gn_weight.reshape(1, D)
    gnb = gn_bias.reshape(1, D)
    gna = gn_alpha.reshape(1, D)
    batch2d = batch.astype(jnp.int32).reshape(N, 1)
    zeros = jnp.zeros((N, D), jnp.float32)

    aggs = []
    for (lo, ne, gather_f, scatter_f, nblk) in (
            (0, EA, _gather_a, _scatter_a, 3),
            (EA, EB, _gather_b, _scatter_b, 2)):
        s = lax.dynamic_slice_in_dim(src, lo, ne)
        d = lax.dynamic_slice_in_dim(dst, lo, ne)
        s_r = s.reshape(NW, ne // NW // GCH, GCH)
        d_r = d.reshape(NW, ne // NW // GCH, GCH)
        hs, hd = gather_f(h, s_r, d_r)
        ea_c = lax.dynamic_slice_in_dim(eab, lo, ne)
        m = _edge_mlp(hd, hs, ea_c, W1d, W1s, W1e, b1, W2b, b2)
        d0 = jnp.where(d < NHALF, d, NHALF).reshape(NS, nblk, IDXC, GCH)
        d1 = jnp.where(d >= NHALF, d - NHALF, NHALF).reshape(NS, nblk, IDXC, GCH)
        aggs.append(scatter_f(m, jnp.stack([d0, d1]), zeros))

    return _update_norm(h, aggs[0], aggs[1], Wu1h, Wu1a, bu1, W_upd2, bu2,
                        batch2d, gnw, gnb, gna)


# speedup vs baseline: 1.0375x; 1.0375x over previous
import functools

import jax
import jax.numpy as jnp
from jax import lax
from jax.experimental import pallas as pl
from jax.experimental.pallas import tpu as pltpu
from jax.experimental.pallas import tpu_sc as plsc

N = 10000
E = 320000
D = 128
DE = 16
DH = 256
G = 64

NC = 2
NS = 16
NW = NC * NS
GCH = 40
K = 5
GRP = K * GCH
NPW = 624
NTAIL = N - NS * NPW

EA = 192000
EB = E - EA

TE = 2000

K2 = 2
GRP2 = K2 * GCH
IDXC = 100
NGB = IDXC // K2

_sc_mesh = plsc.VectorSubcoreMesh(core_axis_name="c", subcore_axis_name="s",
                                  num_cores=NC, num_subcores=NS)


def _make_gather(ne):
    epw = ne // NW
    ndma = epw // GCH
    ng = epw // GRP
    assert ng * GRP == epw and ng % 2 == 0

    @functools.partial(
        pl.kernel,
        out_type=(jax.ShapeDtypeStruct((ne, D), jnp.float32),
                  jax.ShapeDtypeStruct((ne, D), jnp.float32)),
        mesh=_sc_mesh,
        scratch_types=[
            pltpu.VMEM((ndma, GCH), jnp.int32),
            pltpu.VMEM((GRP, D), jnp.float32),
            pltpu.VMEM((GRP, D), jnp.float32),
            pltpu.SemaphoreType.DMA,
            pltpu.SemaphoreType.DMA,
            pltpu.SemaphoreType.DMA,
            pltpu.SemaphoreType.DMA,
        ],
    )
    def gather(h_hbm, src_hbm, dst_hbm, hs_hbm, hd_hbm,
               idx_v, rows0, rows1, sem0, sem1, wsem0, wsem1):
        cid = lax.axis_index("c")
        sid = lax.axis_index("s")
        wid = sid * NC + cid
        base = wid * epw

        for ind_hbm, out_hbm in ((src_hbm, hs_hbm), (dst_hbm, hd_hbm)):
            pltpu.sync_copy(ind_hbm.at[wid], idx_v)

            def fire(g, rows, sem):
                for b in range(K):
                    pltpu.async_copy(h_hbm.at[idx_v.at[g * K + b]],
                                     rows.at[pl.ds(b * GCH, GCH)], sem)

            def drain(rows, sem):
                for b in range(K):
                    pltpu.make_async_copy(h_hbm.at[pl.ds(0, GCH)],
                                          rows.at[pl.ds(b * GCH, GCH)],
                                          sem).wait()

            def process(g, rows, sem, wsem, orows, osem, owsem, fire_guard):
                @pl.when(g >= 1)
                def _():
                    pltpu.make_async_copy(orows, out_hbm.at[pl.ds(base, GRP)],
                                          owsem).wait()
                if fire_guard is None:
                    fire(g + 1, orows, osem)
                else:
                    @pl.when(fire_guard)
                    def _():
                        fire(g + 1, orows, osem)
                drain(rows, sem)
                pltpu.async_copy(rows, out_hbm.at[pl.ds(base + g * GRP, GRP)],
                                 wsem)

            fire(0, rows0, sem0)

            @pl.loop(0, ng, step=2)
            def _(g):
                process(g, rows0, sem0, wsem0, rows1, sem1, wsem1, None)
                process(g + 1, rows1, sem1, wsem1, rows0, sem0, wsem0,
                        g + 2 < ng)

            pltpu.make_async_copy(rows1, out_hbm.at[pl.ds(base, GRP)],
                                  wsem1).wait()

    return gather


_gather_a = _make_gather(EA)
_gather_b = _make_gather(EB)


GCH2 = 80
IDXC2 = 50


def _make_scatter(ne, nblk):
    epw2 = ne // NS
    assert nblk * IDXC2 * GCH2 == epw2

    @functools.partial(
        pl.kernel,
        out_type=jax.ShapeDtypeStruct((N, D), jnp.float32),
        mesh=_sc_mesh,
        scratch_types=[
            pltpu.VMEM((IDXC2, GCH2), jnp.int32),
            pltpu.VMEM((GCH2, D), jnp.float32),
            pltpu.VMEM((GCH2, D), jnp.float32),
            pltpu.VMEM_SHARED((N, D), jnp.float32),
            pltpu.SemaphoreType.DMA,
            pltpu.SemaphoreType.DMA,
            pltpu.SemaphoreType.DMA,
            pltpu.SemaphoreType.DMA,
        ],
    )
    def scatter(m_hbm, dsti_hbm, zeros_hbm, out_hbm,
                idx_v, rows0, rows1, acc, sem0, sem1, ssem0, ssem1):
        cid = lax.axis_index("c")
        sid = lax.axis_index("s")

        @pl.when(cid == 0)
        def _():
            pltpu.sync_copy(zeros_hbm.at[pl.ds(sid * NPW, NPW)],
                            acc.at[pl.ds(sid * NPW, NPW)])

            @pl.when(sid == NS - 1)
            def _():
                pltpu.sync_copy(zeros_hbm.at[pl.ds(NS * NPW, NTAIL)],
                                acc.at[pl.ds(NS * NPW, NTAIL)])

        plsc.subcore_barrier()

        @pl.when(cid == 0)
        def _():
            def fire(bbase, g, rows, sem):
                pltpu.async_copy(m_hbm.at[pl.ds(bbase + g * GCH2, GCH2)],
                                 rows, sem)

            def drain_add(orows, ssem):
                pltpu.make_async_copy(orows, acc.at[pl.ds(0, GCH2)],
                                      ssem).wait()

            def process(bbase, g, rows, sem, ssem_cur, orows, osem, ssem_opp,
                        fire_guard):
                @pl.when(g >= 1)
                def _():
                    drain_add(orows, ssem_opp)
                if fire_guard is None:
                    fire(bbase, g + 1, orows, osem)
                else:
                    @pl.when(fire_guard)
                    def _():
                        fire(bbase, g + 1, orows, osem)
                pltpu.make_async_copy(m_hbm.at[pl.ds(0, GCH2)], rows,
                                      sem).wait()
                pltpu.async_copy(rows, acc.at[idx_v.at[g]], ssem_cur,
                                 add=True)

            for ib in range(nblk):
                pltpu.sync_copy(dsti_hbm.at[sid].at[ib], idx_v)
                bbase = sid * epw2 + ib * (IDXC2 * GCH2)
                fire(bbase, 0, rows0, sem0)

                @pl.loop(0, IDXC2, step=2)
                def _(g):
                    process(bbase, g, rows0, sem0, ssem0, rows1, sem1, ssem1,
                            None)
                    process(bbase, g + 1, rows1, sem1, ssem1, rows0, sem0,
                            ssem0, g + 2 < IDXC2)

                drain_add(rows1, ssem1)

        plsc.subcore_barrier()

        @pl.when(cid == 0)
        def _():
            pltpu.sync_copy(acc.at[pl.ds(sid * NPW, NPW)],
                            out_hbm.at[pl.ds(sid * NPW, NPW)])

            @pl.when(sid == NS - 1)
            def _():
                pltpu.sync_copy(acc.at[pl.ds(NS * NPW, NTAIL)],
                                out_hbm.at[pl.ds(NS * NPW, NTAIL)])

    return scatter


_scatter_a = _make_scatter(EA, 3)
_scatter_b = _make_scatter(EB, 2)


def _edge_mlp_body(hd_ref, hs_ref, ea_ref, w1d_ref, w1s_ref, w1e_ref, b1_ref,
                   w2_ref, b2_ref, m_ref):
    t = (jnp.dot(hd_ref[...].astype(jnp.bfloat16), w1d_ref[...],
                 preferred_element_type=jnp.float32)
         + jnp.dot(hs_ref[...].astype(jnp.bfloat16), w1s_ref[...],
                   preferred_element_type=jnp.float32)
         + jnp.dot(ea_ref[...], w1e_ref[...], preferred_element_type=jnp.float32)
         + b1_ref[...])
    t = t * lax.logistic(t)
    m_ref[...] = jnp.dot(t.astype(jnp.bfloat16), w2_ref[...],
                         preferred_element_type=jnp.float32) + b2_ref[...]


def _edge_mlp(hd, hs, ea, W1d, W1s, W1e, b1, W2, b2):
    ne = hd.shape[0]
    grid = (ne // TE,)
    full = lambda shape: pl.BlockSpec(shape, lambda i: (0, 0))
    return pl.pallas_call(
        _edge_mlp_body,
        grid=grid,
        in_specs=[
            pl.BlockSpec((TE, D), lambda i: (i, 0)),
            pl.BlockSpec((TE, D), lambda i: (i, 0)),
            pl.BlockSpec((TE, DE), lambda i: (i, 0)),
            full((D, DH)), full((D, DH)), full((DE, DH)), full((1, DH)),
            full((DH, D)), full((1, D)),
        ],
        out_specs=pl.BlockSpec((TE, D), lambda i: (i, 0)),
        out_shape=jax.ShapeDtypeStruct((ne, D), jnp.float32),
    )(hd, hs, ea, W1d, W1s, W1e, b1, W2, b2)


def _update_norm_body(h_ref, a0_ref, a1_ref, wu1h_ref, wu1a_ref, bu1_ref,
                      wu2_ref, bu2_ref, batch_ref, gnw_ref, gnb_ref, gna_ref,
                      out_ref):
    h = h_ref[...]
    agg = a0_ref[...] + a1_ref[...]
    t = (jnp.dot(h, wu1h_ref[...], preferred_element_type=jnp.float32)
         + jnp.dot(agg, wu1a_ref[...], preferred_element_type=jnp.float32)
         + bu1_ref[...])
    t = t * lax.logistic(t)
    dh = jnp.dot(t, wu2_ref[...], preferred_element_type=jnp.float32) + bu2_ref[...]
    h2 = h + dh
    oh = (batch_ref[...] == lax.broadcasted_iota(jnp.int32, (1, G), 1)).astype(jnp.float32)
    counts = jnp.maximum(jnp.sum(oh, axis=0, keepdims=True), 1.0)
    inv_counts = jnp.reshape(1.0 / counts, (G, 1))
    seg = lax.dot_general(oh, h2, (((0,), (0,)), ((), ())),
                          preferred_element_type=jnp.float32)
    mean = seg * inv_counts
    out = h2 - gna_ref[...] * jnp.dot(oh, mean, preferred_element_type=jnp.float32)
    var = lax.dot_general(oh, out * out, (((0,), (0,)), ((), ())),
                          preferred_element_type=jnp.float32) * inv_counts
    inv = lax.rsqrt(var + 1e-5)
    out = out * jnp.dot(oh, inv, preferred_element_type=jnp.float32)
    out_ref[...] = gnw_ref[...] * out + gnb_ref[...]


def _update_norm(h, agg0, agg1, Wu1h, Wu1a, bu1, Wu2, bu2, batch2d, gnw, gnb, gna):
    return pl.pallas_call(
        _update_norm_body,
        out_shape=jax.ShapeDtypeStruct((N, D), jnp.float32),
    )(h, agg0, agg1, Wu1h, Wu1a, bu1, Wu2, bu2, batch2d, gnw, gnb, gna)


def kernel(h, edge_index, edge_attr, batch,
           W_msg1, b_msg1, W_msg2, b_msg2,
           W_upd1, b_upd1, W_upd2, b_upd2,
           gn_weight, gn_bias, gn_alpha):
    src = edge_index[0].astype(jnp.int32)
    dst = edge_index[1].astype(jnp.int32)
    W1d = W_msg1[:D].astype(jnp.bfloat16)
    W1s = W_msg1[D:2 * D].astype(jnp.bfloat16)
    W1e = W_msg1[2 * D:].astype(jnp.bfloat16)
    W2b = W_msg2.astype(jnp.bfloat16)
    eab = edge_attr.astype(jnp.bfloat16)
    b1 = b_msg1.reshape(1, DH)
    b2 = b_msg2.reshape(1, D)
    Wu1h = W_upd1[:D]
    Wu1a = W_upd1[D:]
    bu1 = b_upd1.reshape(1, DH)
    bu2 = b_upd2.reshape(1, D)
    gnw = gn_weight.reshape(1, D)
    gnb = gn_bias.reshape(1, D)
    gna = gn_alpha.reshape(1, D)
    batch2d = batch.astype(jnp.int32).reshape(N, 1)
    zeros = jnp.zeros((N, D), jnp.float32)

    aggs = []
    for (lo, ne, gather_f, scatter_f, nblk) in (
            (0, EA, _gather_a, _scatter_a, 3),
            (EA, EB, _gather_b, _scatter_b, 2)):
        s = lax.dynamic_slice_in_dim(src, lo, ne)
        d = lax.dynamic_slice_in_dim(dst, lo, ne)
        s_r = s.reshape(NW, ne // NW // GCH, GCH)
        d_r = d.reshape(NW, ne // NW // GCH, GCH)
        hs, hd = gather_f(h, s_r, d_r)
        ea_c = lax.dynamic_slice_in_dim(eab, lo, ne)
        m = _edge_mlp(hd, hs, ea_c, W1d, W1s, W1e, b1, W2b, b2)
        aggs.append(scatter_f(m, d.reshape(NS, nblk, IDXC2, GCH2), zeros))

    return _update_norm(h, aggs[0], aggs[1], Wu1h, Wu1a, bu1, W_upd2, bu2,
                        batch2d, gnw, gnb, gna)
